# baseline (speedup 1.0000x reference)
import functools

import jax
import jax.numpy as jnp
from jax import lax
from jax.experimental import pallas as pl
from jax.experimental.pallas import tpu as pltpu
from jax.experimental.pallas import tpu_sc as plsc

B, C, V_LVL, V_PREV, PATCH = 8, 128, 40962, 10242, 7
ROWS = B * C
NW = 32
ROWS_PER_W = ROWS // NW
N_PAIRS = (V_PREV + 31) // 32
WPJ = N_PAIRS * 16
COMB_W = PATCH * WPJ
XROW_PAD = 40968
OUT4 = 4 * V_PREV
OUT4_PAD = OUT4 + 32
FLAT_P = V_PREV * PATCH
PB = 1440
PAIRS_PB = PB // 32
NB = 7
TAIL_P0 = NB * PB
TAIL_PAIRS = N_PAIRS - NB * PAIRS_PB
IMAX = V_LVL - 1


@functools.partial(
    pl.kernel,
    out_type=jax.ShapeDtypeStruct((ROWS * V_PREV,), jnp.float32),
    mesh=plsc.VectorSubcoreMesh(core_axis_name="c", subcore_axis_name="s"),
    compiler_params=pltpu.CompilerParams(needs_layout_passes=False),
    scratch_types=[
        pltpu.VMEM((COMB_W,), jnp.int32),
        pltpu.VMEM((XROW_PAD,), jnp.float32),
        pltpu.VMEM((OUT4_PAD,), jnp.float32),
    ],
)
def _sc_maxpool(x_hbm, v2p_hbm, patches_hbm, out_hbm, comb_v, xrow_v, out4_v):
    wid = lax.axis_index("s") * 2 + lax.axis_index("c")
    lane7 = lax.iota(jnp.int32, 16) * 7

    pltpu.sync_copy(v2p_hbm, xrow_v.at[pl.ds(0, V_LVL)])

    def compose_batch(b, n_pairs, p0, win):
        pltpu.sync_copy(patches_hbm.at[pl.ds(p0 * PATCH, win)],
                        out4_v.at[pl.ds(0, win)])

        def pair(c2l, carry):
            fbase = c2l * (32 * PATCH)

            def compose_half(f0):
                pidx = plsc.bitcast(plsc.load_gather(out4_v, [f0]), jnp.int32)
                pidx = jnp.clip(pidx, 0, IMAX)
                v = plsc.bitcast(plsc.load_gather(xrow_v, [pidx]), jnp.int32)
                return jnp.clip(v, 0, IMAX)

            for j in range(PATCH):
                fe = lane7 + (fbase + j)
                ve = compose_half(fe)
                vo = compose_half(fe + 16 * PATCH)
                word = jnp.bitwise_or(ve, lax.shift_left(vo, 16))
                c2g = b * PAIRS_PB + c2l
                comb_v[pl.ds(j * WPJ + c2g * 16, 16)] = word
            return carry

        lax.fori_loop(0, n_pairs, pair, 0)

    for b in range(NB):
        compose_batch(b, PAIRS_PB, b * PB, PB * PATCH)
    compose_batch(NB, TAIL_PAIRS, TAIL_P0, FLAT_P - TAIL_P0 * PATCH)

    def do_group(g, carry):
        for i4 in range(4):
            r = wid * ROWS_PER_W + g * 4 + i4
            base = r * V_LVL
            delta = lax.bitwise_and(base, 7)
            a = pl.multiple_of(base - delta, 8)
            pltpu.sync_copy(x_hbm.at[pl.ds(a, XROW_PAD)], xrow_v)

            def pair(c2, carry2):
                me = None
                mo = None
                for j in range(PATCH):
                    w = comb_v[pl.ds(j * WPJ + c2 * 16, 16)]
                    ae = jnp.bitwise_and(w, 0xFFFF) + delta
                    ao = lax.shift_right_logical(w, 16) + delta
                    ge = plsc.load_gather(xrow_v, [ae])
                    go = plsc.load_gather(xrow_v, [ao])
                    me = ge if me is None else jnp.maximum(me, ge)
                    mo = go if mo is None else jnp.maximum(mo, go)
                off = i4 * V_PREV + c2 * 32
                out4_v[pl.ds(off, 16)] = me
                out4_v[pl.ds(off + 16, 16)] = mo
                return carry2

            lax.fori_loop(0, N_PAIRS, pair, 0)

        off = (wid * 8 + g) * OUT4
        pltpu.sync_copy(out4_v.at[pl.ds(0, OUT4)],
                        out_hbm.at[pl.ds(pl.multiple_of(off, 8), OUT4)])
        return carry

    lax.fori_loop(0, ROWS_PER_W // 4, do_group, 0)


def kernel(x, vertices_to_prev_lvl, neihboring_patches):
    x_flat = x.reshape(-1)
    v2p_f = lax.bitcast_convert_type(vertices_to_prev_lvl, jnp.float32)
    patches_f = lax.bitcast_convert_type(
        neihboring_patches.reshape(-1), jnp.float32)
    out_flat = _sc_maxpool(x_flat, v2p_f, patches_f)
    return out_flat.reshape(B, C, V_PREV)
